# baseline (speedup 1.0000x reference)
import functools
import numpy as np
import jax
import jax.numpy as jnp
from jax.experimental import pallas as pl
from jax.experimental.pallas import tpu as pltpu

BLOCK_NUM = 1024
C = 128
HS = [224, 112, 56, 28]


def _upsample_weights(h, H):
    src = (np.arange(H) + 0.5) * h / H - 0.5
    r0 = np.floor(src).astype(np.int32)
    w1 = (src - r0).astype(np.float32)
    r1 = np.clip(r0 + 1, 0, h - 1).astype(np.int32)
    r0 = np.clip(r0, 0, h - 1).astype(np.int32)
    return r0, r1, (1.0 - w1), w1


def _graph_mm_body(adj_ref, nodes_ref, cnts_ref, w_ref, g_ref, cnt0_ref):
    adj = adj_ref[...]
    a = adj + jnp.eye(BLOCK_NUM, dtype=jnp.float32)
    d = jnp.sum(a, axis=1)
    dinv = jax.lax.rsqrt(jnp.clip(d, 1e-6, None))
    an = a * dinv[:, None] * dinv[None, :]

    cnts = jnp.maximum(cnts_ref[...], 1.0)
    p = []
    for i in range(4):
        node = nodes_ref[i] / cnts[i][:, None]
        p.append(jnp.dot(node, w_ref[i], preferred_element_type=jnp.float32))
    pcat = jnp.concatenate(p, axis=1)
    g_ref[...] = jnp.dot(an, pcat, preferred_element_type=jnp.float32)
    cnt0_ref[...] = cnts_ref[0:1, :]


def _graph_mm(adj, nodes, cnts, ws):
    return pl.pallas_call(
        _graph_mm_body,
        out_shape=(
            jax.ShapeDtypeStruct((BLOCK_NUM, 4 * C), jnp.float32),
            jax.ShapeDtypeStruct((1, BLOCK_NUM), jnp.float32),
        ),
    )(adj, nodes, cnts, ws)


def _final_mm_body(adj_ref, b0_ref, ybs_ref, cnt0_ref, wf0_ref, gf_ref):
    adj = adj_ref[...]
    a = adj + jnp.eye(BLOCK_NUM, dtype=jnp.float32)
    d = jnp.sum(a, axis=1)
    dinv = jax.lax.rsqrt(jnp.clip(d, 1e-6, None))
    an = a * dinv[:, None] * dinv[None, :]
    t = jnp.dot(b0_ref[...], wf0_ref[...], preferred_element_type=jnp.float32)
    t = (t + ybs_ref[...]) / jnp.maximum(cnt0_ref[...], 1.0).reshape(BLOCK_NUM, 1)
    gf_ref[...] = jnp.dot(an, t, preferred_element_type=jnp.float32)


def _final_mm(adj, b0, ybs, cnt0, wf0):
    return pl.pallas_call(
        _final_mm_body,
        out_shape=jax.ShapeDtypeStruct((BLOCK_NUM, 16), jnp.float32),
    )(adj, b0, ybs, cnt0, wf0)


def _maxpool(v):
    vp = jnp.pad(v, ((1, 1), (1, 1), (0, 0)), constant_values=-np.inf)
    m = jnp.maximum(jnp.maximum(vp[:-2], vp[1:-1]), vp[2:])
    m = jnp.maximum(jnp.maximum(m[:, :-2], m[:, 1:-1]), m[:, 2:])
    return m[::2, ::2]


def kernel(x, index, adj, W0, W1, W2, W3, Wf):
    idxs = [index.astype(jnp.int32)[::s, ::s].reshape(-1) for s in (1, 2, 4, 8)]
    ws = jnp.stack([W0, W1, W2, W3])

    xp = [x]
    for _ in range(3):
        xp.append(_maxpool(xp[-1]))

    nodes, cnts = [], []
    for i in range(4):
        feats = xp[i].reshape(-1, C)
        nodes.append(jax.ops.segment_sum(feats, idxs[i], num_segments=BLOCK_NUM))
        cnts.append(jax.ops.segment_sum(jnp.ones((feats.shape[0],), jnp.float32),
                                        idxs[i], num_segments=BLOCK_NUM))

    g, cnt0 = _graph_mm(adj, jnp.stack(nodes), jnp.stack(cnts), ws)
    cnt0 = cnt0.reshape(-1)

    u = jax.nn.relu(jnp.take(g[:, :C], idxs[0], axis=0) + x.reshape(-1, C))
    b0 = jax.ops.segment_sum(u, idxs[0], num_segments=BLOCK_NUM)

    yB = jnp.zeros((224, 224, 16), jnp.float32)
    for i in range(1, 4):
        h = HS[i]
        f = jax.nn.relu(jnp.take(g[:, C * i:C * (i + 1)], idxs[i], axis=0)
                        + xp[i].reshape(-1, C))
        yi = (f @ Wf[C * i:C * (i + 1)]).reshape(h, h, 16)
        r0, r1, a0, a1 = _upsample_weights(h, 224)
        rows = yi[r0] * a0[:, None, None] + yi[r1] * a1[:, None, None]
        yB = yB + rows[:, r0] * a0[None, :, None] + rows[:, r1] * a1[None, :, None]

    ybs = jax.ops.segment_sum(yB.reshape(-1, 16), idxs[0], num_segments=BLOCK_NUM)
    gf = _final_mm(adj, b0, ybs, cnt0, Wf[:C])

    finall = jnp.take(gf, idxs[0], axis=0).T.reshape(1, 16, 224, 224)
    sm = jax.nn.softmax(finall, axis=1)
    return finall, sm


# speedup vs baseline: 1.3213x; 1.3213x over previous
import functools
import numpy as np
import jax
import jax.numpy as jnp
from jax import lax
from jax.experimental import pallas as pl
from jax.experimental.pallas import tpu as pltpu
from jax.experimental.pallas import tpu_sc as plsc

BLOCK_NUM = 1024
C = 128
HS = [224, 112, 56, 28]
NC, NS, NW = 2, 16, 32
CH = 112
ACC_R = 1152


_SEG_SPECS = [
    (50176, 14),
    (14336, 4),
    (3584, 1),
    (3584, 1),
]


def _segsum_body(f0, f1, f2, f3, i0, i1, i2, i3, v0, v1, v2, v3, z128, zhist,
                 sums, cnts, a0, a1, a2, a3, xbuf, idxbuf, idxvb, hist):
    c = lax.axis_index("c")
    s = lax.axis_index("s")
    w = s * NC + c
    accs = [a0, a1, a2, a3]
    feats = [f0, f1, f2, f3]
    idxs = [i0, i1, i2, i3]
    idxvs = [v0, v1, v2, v3]
    lane = jnp.arange(16, dtype=jnp.int32)
    one16 = jnp.ones((16,), jnp.float32)
    zero16 = jnp.zeros((16,), jnp.float32)
    pltpu.sync_copy(z128, xbuf.at[pl.ds(0, 72)])
    for k in range(4):
        pltpu.sync_copy(xbuf.at[pl.ds(0, 72)], accs[k].at[pl.ds(s * 72, 72)])
    plsc.subcore_barrier()
    for k, (_, chunks) in enumerate(_SEG_SPECS):
        per_w = chunks * CH
        pltpu.sync_copy(zhist, hist)
        pltpu.sync_copy(idxs[k].at[w], idxbuf.at[pl.ds(0, chunks)])
        pltpu.sync_copy(idxvs[k].at[w], idxvb.at[pl.ds(0, chunks * 7)])

        def cbody(t, _):
            flat = lane * 1040 + idxvb[t]
            plsc.addupdate_scatter(hist, [flat], one16)
            return 0

        lax.fori_loop(0, chunks * 7, cbody, 0)
        for j in range(chunks):
            p0 = w * per_w + j * CH
            pltpu.sync_copy(feats[k].at[pl.ds(p0, CH), :], xbuf)
            pltpu.sync_copy(xbuf, accs[k].at[idxbuf.at[j]], add=True)
        pltpu.sync_copy(hist, cnts.at[k, w])
    plsc.subcore_barrier()
    for k in range(4):
        pltpu.sync_copy(accs[k].at[pl.ds(s * 64, 64)], xbuf.at[pl.ds(0, 64)])
        pltpu.sync_copy(xbuf.at[pl.ds(0, 64)], sums.at[k, c, pl.ds(s * 64, 64)])


def _node_segsums(feats_list, idx_list):
    mesh = plsc.VectorSubcoreMesh(core_axis_name="c", subcore_axis_name="s", num_cores=NC, num_subcores=NS)
    scratch = ([pltpu.VMEM_SHARED((ACC_R, C), jnp.float32)] * 4
               + [pltpu.VMEM((CH, C), jnp.float32),
                  pltpu.VMEM((14, CH), jnp.int32),
                  pltpu.VMEM((98, 16), jnp.int32),
                  pltpu.VMEM((16640,), jnp.float32)])
    fn = pl.kernel(
        _segsum_body,
        out_type=(jax.ShapeDtypeStruct((4, NC, BLOCK_NUM, C), jnp.float32),
                  jax.ShapeDtypeStruct((4, NW, 16640), jnp.float32)),
        mesh=mesh,
        compiler_params=pltpu.CompilerParams(needs_layout_passes=False),
        scratch_types=scratch,
    )
    z128 = jnp.zeros((72, C), jnp.float32)
    zhist = jnp.zeros((16640,), jnp.float32)
    idxv_list = [a.reshape(NW, -1, 16) for a in idx_list]
    return fn(*feats_list, *idx_list, *idxv_list, z128, zhist)


def _b0_body(xf, i0, g0, z128, b0out, acc, g0s, xbuf, gbuf, idxbuf, zbuf, obuf):
    c = lax.axis_index("c")
    s = lax.axis_index("s")
    w = s * NC + c
    pltpu.sync_copy(g0.at[pl.ds(s * 64, 64)], obuf)
    pltpu.sync_copy(obuf, g0s.at[pl.ds(s * 64, 64)])
    pltpu.sync_copy(z128, zbuf)
    pltpu.sync_copy(zbuf, acc.at[pl.ds(s * 72, 72)])
    pltpu.sync_copy(i0.at[w], idxbuf)
    plsc.subcore_barrier()
    for j in range(14):
        p0 = w * 1568 + j * CH
        pltpu.sync_copy(xf.at[pl.ds(p0, CH), :], xbuf)
        pltpu.sync_copy(g0s.at[idxbuf.at[j]], gbuf)

        def row_body(i, _):
            for cc in range(8):
                v = xbuf[i, pl.ds(cc * 16, 16)] + gbuf[i, pl.ds(cc * 16, 16)]
                xbuf[i, pl.ds(cc * 16, 16)] = jnp.maximum(v, 0.0)
            return 0

        lax.fori_loop(0, CH, row_body, 0)
        pltpu.sync_copy(xbuf, acc.at[idxbuf.at[j]], add=True)
    plsc.subcore_barrier()
    pltpu.sync_copy(acc.at[pl.ds(s * 64, 64)], obuf)
    pltpu.sync_copy(obuf, b0out.at[c, pl.ds(s * 64, 64)])


def _block0_segsum(xf, i0_3d, g0):
    mesh = plsc.VectorSubcoreMesh(core_axis_name="c", subcore_axis_name="s", num_cores=NC, num_subcores=NS)
    fn = pl.kernel(
        _b0_body,
        out_type=jax.ShapeDtypeStruct((NC, BLOCK_NUM, C), jnp.float32),
        mesh=mesh,
        compiler_params=pltpu.CompilerParams(needs_layout_passes=False),
        scratch_types=[
            pltpu.VMEM_SHARED((ACC_R, C), jnp.float32),
            pltpu.VMEM_SHARED((BLOCK_NUM, C), jnp.float32),
            pltpu.VMEM((CH, C), jnp.float32),
            pltpu.VMEM((CH, C), jnp.float32),
            pltpu.VMEM((14, CH), jnp.int32),
            pltpu.VMEM((72, C), jnp.float32),
            pltpu.VMEM((64, C), jnp.float32),
        ],
    )
    return fn(xf, i0_3d, g0, jnp.zeros((72, C), jnp.float32))


def _fin_body(gf, iv, out3, gfs, idxvb, outb):
    c = lax.axis_index("c")
    s = lax.axis_index("s")
    w = s * NC + c
    pltpu.sync_copy(gf, gfs)
    pltpu.sync_copy(iv.at[w], idxvb)

    def grp_body(t, _):
        idxv = idxvb[t]
        for ch in range(16):
            vals = plsc.load_gather(gfs, [idxv, jnp.full((16,), ch, jnp.int32)])
            outb[ch, t, :] = vals
        return 0

    lax.fori_loop(0, 98, grp_body, 0)
    pltpu.sync_copy(outb, out3.at[w])


def _final_gather(gf, i0_3d):
    mesh = plsc.VectorSubcoreMesh(core_axis_name="c", subcore_axis_name="s", num_cores=NC, num_subcores=NS)
    fn = pl.kernel(
        _fin_body,
        out_type=jax.ShapeDtypeStruct((NW, 16, 98, 16), jnp.float32),
        mesh=mesh,
        compiler_params=pltpu.CompilerParams(needs_layout_passes=False),
        scratch_types=[
            pltpu.VMEM((BLOCK_NUM, 16), jnp.float32),
            pltpu.VMEM((98, 16), jnp.int32),
            pltpu.VMEM((16, 98, 16), jnp.float32),
        ],
    )
    return fn(gf, i0_3d.reshape(NW, 98, 16))


def _pad_rows(arr, n, value=0):
    return jnp.pad(arr, ((0, n - arr.shape[0]),) + ((0, 0),) * (arr.ndim - 1),
                   constant_values=value)


def _upsample_weights(h, H):
    src = (np.arange(H) + 0.5) * h / H - 0.5
    r0 = np.floor(src).astype(np.int32)
    w1 = (src - r0).astype(np.float32)
    r1 = np.clip(r0 + 1, 0, h - 1).astype(np.int32)
    r0 = np.clip(r0, 0, h - 1).astype(np.int32)
    return r0, r1, (1.0 - w1), w1


def _graph_mm_body(adj_ref, nodes_ref, cnts_ref, w_ref, g_ref, cnt0_ref):
    adj = adj_ref[...]
    a = adj + jnp.eye(BLOCK_NUM, dtype=jnp.float32)
    d = jnp.sum(a, axis=1)
    dinv = jax.lax.rsqrt(jnp.clip(d, 1e-6, None))
    an = a * dinv[:, None] * dinv[None, :]

    cnts = jnp.maximum(cnts_ref[...], 1.0)
    p = []
    for i in range(4):
        node = nodes_ref[i] / cnts[i][:, None]
        p.append(jnp.dot(node, w_ref[i], preferred_element_type=jnp.float32))
    pcat = jnp.concatenate(p, axis=1)
    g_ref[...] = jnp.dot(an, pcat, preferred_element_type=jnp.float32)
    cnt0_ref[...] = cnts_ref[0:1, :]


def _graph_mm(adj, nodes, cnts, ws):
    return pl.pallas_call(
        _graph_mm_body,
        out_shape=(
            jax.ShapeDtypeStruct((BLOCK_NUM, 4 * C), jnp.float32),
            jax.ShapeDtypeStruct((1, BLOCK_NUM), jnp.float32),
        ),
    )(adj, nodes, cnts, ws)


def _final_mm_body(adj_ref, b0_ref, ybs_ref, cnt0_ref, wf0_ref, gf_ref):
    adj = adj_ref[...]
    a = adj + jnp.eye(BLOCK_NUM, dtype=jnp.float32)
    d = jnp.sum(a, axis=1)
    dinv = jax.lax.rsqrt(jnp.clip(d, 1e-6, None))
    an = a * dinv[:, None] * dinv[None, :]
    t = jnp.dot(b0_ref[...], wf0_ref[...], preferred_element_type=jnp.float32)
    t = (t + ybs_ref[...]) / jnp.maximum(cnt0_ref[...], 1.0).reshape(BLOCK_NUM, 1)
    gf_ref[...] = jnp.dot(an, t, preferred_element_type=jnp.float32)


def _final_mm(adj, b0, ybs, cnt0, wf0):
    return pl.pallas_call(
        _final_mm_body,
        out_shape=jax.ShapeDtypeStruct((BLOCK_NUM, 16), jnp.float32),
    )(adj, b0, ybs, cnt0, wf0)


def _maxpool(v):
    vp = jnp.pad(v, ((1, 1), (1, 1), (0, 0)), constant_values=-np.inf)
    m = jnp.maximum(jnp.maximum(vp[:-2], vp[1:-1]), vp[2:])
    m = jnp.maximum(jnp.maximum(m[:, :-2], m[:, 1:-1]), m[:, 2:])
    return m[::2, ::2]


def kernel(x, index, adj, W0, W1, W2, W3, Wf):
    idxs = [index.astype(jnp.int32)[::s, ::s].reshape(-1) for s in (1, 2, 4, 8)]
    ws = jnp.stack([W0, W1, W2, W3])

    xp = [x]
    for _ in range(3):
        xp.append(_maxpool(xp[-1]))

    feats_list, idx_list = [], []
    for i in range(4):
        npad = _SEG_SPECS[i][0]
        feats_list.append(_pad_rows(xp[i].reshape(-1, C), npad))
        idx_list.append(_pad_rows(idxs[i], npad, BLOCK_NUM).reshape(NW, -1, CH))
    sums, scnts = _node_segsums(feats_list, idx_list)
    nodes = jnp.sum(sums, axis=1)
    cnt4 = jnp.sum(scnts.reshape(4, NW * 16, 1040), axis=1)[:, :BLOCK_NUM]

    g, cnt0 = _graph_mm(adj, nodes, cnt4, ws)
    cnt0 = cnt0.reshape(-1)

    u = jax.nn.relu(jnp.take(g[:, :C], idxs[0], axis=0) + x.reshape(-1, C))
    b0 = jax.ops.segment_sum(u, idxs[0], num_segments=BLOCK_NUM)

    yB = jnp.zeros((224, 224, 16), jnp.float32)
    for i in range(1, 4):
        h = HS[i]
        f = jax.nn.relu(jnp.take(g[:, C * i:C * (i + 1)], idxs[i], axis=0)
                        + xp[i].reshape(-1, C))
        yi = (f @ Wf[C * i:C * (i + 1)]).reshape(h, h, 16)
        r0, r1, a0, a1 = _upsample_weights(h, 224)
        rows = yi[r0] * a0[:, None, None] + yi[r1] * a1[:, None, None]
        yB = yB + rows[:, r0] * a0[None, :, None] + rows[:, r1] * a1[None, :, None]

    ybs = jax.ops.segment_sum(yB.reshape(-1, 16), idxs[0], num_segments=BLOCK_NUM)
    gf = _final_mm(adj, b0, ybs, cnt0, Wf[:C])

    finall = jnp.take(gf, idxs[0], axis=0).T.reshape(1, 16, 224, 224)
    sm = jax.nn.softmax(finall, axis=1)
    return finall, sm


# speedup vs baseline: 1.3221x; 1.0006x over previous
import functools
import numpy as np
import jax
import jax.numpy as jnp
from jax import lax
from jax.experimental import pallas as pl
from jax.experimental.pallas import tpu as pltpu
from jax.experimental.pallas import tpu_sc as plsc

BLOCK_NUM = 1024
C = 128
HS = [224, 112, 56, 28]
NC, NS, NW = 2, 16, 32
CH = 112
ACC_R = 1152


_SEG_SPECS = [
    (50176, 14),
    (14336, 4),
    (3584, 1),
    (3584, 1),
]


def _segsum_body(f0, f1, f2, f3, i0, i1, i2, i3, v0, v1, v2, v3, z128, zhist,
                 sums, cnts, a0, a1, a2, a3, xbuf, idxbuf, idxvb, hist):
    c = lax.axis_index("c")
    s = lax.axis_index("s")
    w = s * NC + c
    accs = [a0, a1, a2, a3]
    feats = [f0, f1, f2, f3]
    idxs = [i0, i1, i2, i3]
    idxvs = [v0, v1, v2, v3]
    lane = jnp.arange(16, dtype=jnp.int32)
    one16 = jnp.ones((16,), jnp.float32)
    zero16 = jnp.zeros((16,), jnp.float32)
    pltpu.sync_copy(z128, xbuf.at[pl.ds(0, 72)])
    for k in range(4):
        pltpu.sync_copy(xbuf.at[pl.ds(0, 72)], accs[k].at[pl.ds(s * 72, 72)])
    plsc.subcore_barrier()
    for k, (_, chunks) in enumerate(_SEG_SPECS):
        per_w = chunks * CH
        pltpu.sync_copy(zhist, hist)
        pltpu.sync_copy(idxs[k].at[w], idxbuf.at[pl.ds(0, chunks)])
        pltpu.sync_copy(idxvs[k].at[w], idxvb.at[pl.ds(0, chunks * 7)])

        def cbody(t, _):
            flat = lane * 1040 + idxvb[t]
            plsc.addupdate_scatter(hist, [flat], one16)
            return 0

        lax.fori_loop(0, chunks * 7, cbody, 0)
        for j in range(chunks):
            p0 = w * per_w + j * CH
            pltpu.sync_copy(feats[k].at[pl.ds(p0, CH), :], xbuf)
            pltpu.sync_copy(xbuf, accs[k].at[idxbuf.at[j]], add=True)
        pltpu.sync_copy(hist, cnts.at[k, w])
    plsc.subcore_barrier()
    for k in range(4):
        pltpu.sync_copy(accs[k].at[pl.ds(s * 64, 64)], xbuf.at[pl.ds(0, 64)])
        pltpu.sync_copy(xbuf.at[pl.ds(0, 64)], sums.at[k, c, pl.ds(s * 64, 64)])


def _node_segsums(feats_list, idx_list):
    mesh = plsc.VectorSubcoreMesh(core_axis_name="c", subcore_axis_name="s", num_cores=NC, num_subcores=NS)
    scratch = ([pltpu.VMEM_SHARED((ACC_R, C), jnp.float32)] * 4
               + [pltpu.VMEM((CH, C), jnp.float32),
                  pltpu.VMEM((14, CH), jnp.int32),
                  pltpu.VMEM((98, 16), jnp.int32),
                  pltpu.VMEM((16640,), jnp.float32)])
    fn = pl.kernel(
        _segsum_body,
        out_type=(jax.ShapeDtypeStruct((4, NC, BLOCK_NUM, C), jnp.float32),
                  jax.ShapeDtypeStruct((4, NW, 16640), jnp.float32)),
        mesh=mesh,
        compiler_params=pltpu.CompilerParams(needs_layout_passes=False),
        scratch_types=scratch,
    )
    z128 = jnp.zeros((72, C), jnp.float32)
    zhist = jnp.zeros((16640,), jnp.float32)
    idxv_list = [a.reshape(NW, -1, 16) for a in idx_list]
    return fn(*feats_list, *idx_list, *idxv_list, z128, zhist)


def _b0_body(xf, i0, g0, z128, b0out, acc, g0s, xbuf, gbuf, idxbuf, zbuf, obuf):
    c = lax.axis_index("c")
    s = lax.axis_index("s")
    w = s * NC + c
    pltpu.sync_copy(g0.at[pl.ds(s * 64, 64)], obuf)
    pltpu.sync_copy(obuf, g0s.at[pl.ds(s * 64, 64)])
    pltpu.sync_copy(z128, zbuf)
    pltpu.sync_copy(zbuf, acc.at[pl.ds(s * 72, 72)])
    pltpu.sync_copy(i0.at[w], idxbuf)
    plsc.subcore_barrier()
    for j in range(14):
        p0 = w * 1568 + j * CH
        pltpu.sync_copy(xf.at[pl.ds(p0, CH), :], xbuf)
        pltpu.sync_copy(g0s.at[idxbuf.at[j]], gbuf)

        def row_body(i, _):
            for cc in range(8):
                v = xbuf[i, pl.ds(cc * 16, 16)] + gbuf[i, pl.ds(cc * 16, 16)]
                xbuf[i, pl.ds(cc * 16, 16)] = jnp.maximum(v, 0.0)
            return 0

        lax.fori_loop(0, CH, row_body, 0)
        pltpu.sync_copy(xbuf, acc.at[idxbuf.at[j]], add=True)
    plsc.subcore_barrier()
    pltpu.sync_copy(acc.at[pl.ds(s * 64, 64)], obuf)
    pltpu.sync_copy(obuf, b0out.at[c, pl.ds(s * 64, 64)])


def _block0_segsum(xf, i0_3d, g0):
    mesh = plsc.VectorSubcoreMesh(core_axis_name="c", subcore_axis_name="s", num_cores=NC, num_subcores=NS)
    fn = pl.kernel(
        _b0_body,
        out_type=jax.ShapeDtypeStruct((NC, BLOCK_NUM, C), jnp.float32),
        mesh=mesh,
        compiler_params=pltpu.CompilerParams(needs_layout_passes=False),
        scratch_types=[
            pltpu.VMEM_SHARED((ACC_R, C), jnp.float32),
            pltpu.VMEM_SHARED((BLOCK_NUM, C), jnp.float32),
            pltpu.VMEM((CH, C), jnp.float32),
            pltpu.VMEM((CH, C), jnp.float32),
            pltpu.VMEM((14, CH), jnp.int32),
            pltpu.VMEM((72, C), jnp.float32),
            pltpu.VMEM((64, C), jnp.float32),
        ],
    )
    return fn(xf, i0_3d, g0, jnp.zeros((72, C), jnp.float32))


def _fin_body(gf, iv, out3, gfs, idxvb, outb):
    c = lax.axis_index("c")
    s = lax.axis_index("s")
    w = s * NC + c
    pltpu.sync_copy(gf, gfs)
    pltpu.sync_copy(iv.at[w], idxvb)

    def grp_body(t, _):
        idxv = idxvb[t]
        for ch in range(16):
            vals = plsc.load_gather(gfs, [idxv, jnp.full((16,), ch, jnp.int32)])
            outb[ch, pl.ds(t * 16, 16)] = vals
        return 0

    lax.fori_loop(0, 98, grp_body, 0)
    pltpu.sync_copy(outb, out3.at[w])


def _final_gather(gf, i0_3d):
    mesh = plsc.VectorSubcoreMesh(core_axis_name="c", subcore_axis_name="s", num_cores=NC, num_subcores=NS)
    fn = pl.kernel(
        _fin_body,
        out_type=jax.ShapeDtypeStruct((NW, 16, 1568), jnp.float32),
        mesh=mesh,
        compiler_params=pltpu.CompilerParams(needs_layout_passes=False),
        scratch_types=[
            pltpu.VMEM((BLOCK_NUM, 16), jnp.float32),
            pltpu.VMEM((98, 16), jnp.int32),
            pltpu.VMEM((16, 1568), jnp.float32),
        ],
    )
    return fn(gf, i0_3d.reshape(NW, 98, 16))


def _pad_rows(arr, n, value=0):
    return jnp.pad(arr, ((0, n - arr.shape[0]),) + ((0, 0),) * (arr.ndim - 1),
                   constant_values=value)


def _upsample_weights(h, H):
    src = (np.arange(H) + 0.5) * h / H - 0.5
    r0 = np.floor(src).astype(np.int32)
    w1 = (src - r0).astype(np.float32)
    r1 = np.clip(r0 + 1, 0, h - 1).astype(np.int32)
    r0 = np.clip(r0, 0, h - 1).astype(np.int32)
    return r0, r1, (1.0 - w1), w1


def _graph_mm_body(adj_ref, nodes_ref, cnts_ref, w_ref, g_ref, cnt0_ref):
    adj = adj_ref[...]
    a = adj + jnp.eye(BLOCK_NUM, dtype=jnp.float32)
    d = jnp.sum(a, axis=1)
    dinv = jax.lax.rsqrt(jnp.clip(d, 1e-6, None))
    an = a * dinv[:, None] * dinv[None, :]

    cnts = jnp.maximum(cnts_ref[...], 1.0)
    p = []
    for i in range(4):
        node = nodes_ref[i] / cnts[i][:, None]
        p.append(jnp.dot(node, w_ref[i], preferred_element_type=jnp.float32))
    pcat = jnp.concatenate(p, axis=1)
    g_ref[...] = jnp.dot(an, pcat, preferred_element_type=jnp.float32)
    cnt0_ref[...] = cnts_ref[0:1, :]


def _graph_mm(adj, nodes, cnts, ws):
    return pl.pallas_call(
        _graph_mm_body,
        out_shape=(
            jax.ShapeDtypeStruct((BLOCK_NUM, 4 * C), jnp.float32),
            jax.ShapeDtypeStruct((1, BLOCK_NUM), jnp.float32),
        ),
    )(adj, nodes, cnts, ws)


def _final_mm_body(adj_ref, b0_ref, ybs_ref, cnt0_ref, wf0_ref, gf_ref):
    adj = adj_ref[...]
    a = adj + jnp.eye(BLOCK_NUM, dtype=jnp.float32)
    d = jnp.sum(a, axis=1)
    dinv = jax.lax.rsqrt(jnp.clip(d, 1e-6, None))
    an = a * dinv[:, None] * dinv[None, :]
    t = jnp.dot(b0_ref[...], wf0_ref[...], preferred_element_type=jnp.float32)
    t = (t + ybs_ref[...]) / jnp.maximum(cnt0_ref[...], 1.0).reshape(BLOCK_NUM, 1)
    gf_ref[...] = jnp.dot(an, t, preferred_element_type=jnp.float32)


def _final_mm(adj, b0, ybs, cnt0, wf0):
    return pl.pallas_call(
        _final_mm_body,
        out_shape=jax.ShapeDtypeStruct((BLOCK_NUM, 16), jnp.float32),
    )(adj, b0, ybs, cnt0, wf0)


def _maxpool(v):
    vp = jnp.pad(v, ((1, 1), (1, 1), (0, 0)), constant_values=-np.inf)
    m = jnp.maximum(jnp.maximum(vp[:-2], vp[1:-1]), vp[2:])
    m = jnp.maximum(jnp.maximum(m[:, :-2], m[:, 1:-1]), m[:, 2:])
    return m[::2, ::2]


def kernel(x, index, adj, W0, W1, W2, W3, Wf):
    idxs = [index.astype(jnp.int32)[::s, ::s].reshape(-1) for s in (1, 2, 4, 8)]
    ws = jnp.stack([W0, W1, W2, W3])

    xp = [x]
    for _ in range(3):
        xp.append(_maxpool(xp[-1]))

    feats_list, idx_list = [], []
    for i in range(4):
        npad = _SEG_SPECS[i][0]
        feats_list.append(_pad_rows(xp[i].reshape(-1, C), npad))
        idx_list.append(_pad_rows(idxs[i], npad, BLOCK_NUM).reshape(NW, -1, CH))
    sums, scnts = _node_segsums(feats_list, idx_list)
    nodes = jnp.sum(sums, axis=1)
    cnt4 = jnp.sum(scnts.reshape(4, NW * 16, 1040), axis=1)[:, :BLOCK_NUM]

    g, cnt0 = _graph_mm(adj, nodes, cnt4, ws)
    cnt0 = cnt0.reshape(-1)

    u = jax.nn.relu(jnp.take(g[:, :C], idxs[0], axis=0) + x.reshape(-1, C))
    b0 = jax.ops.segment_sum(u, idxs[0], num_segments=BLOCK_NUM)

    yB = jnp.zeros((224, 224, 16), jnp.float32)
    for i in range(1, 4):
        h = HS[i]
        f = jax.nn.relu(jnp.take(g[:, C * i:C * (i + 1)], idxs[i], axis=0)
                        + xp[i].reshape(-1, C))
        yi = (f @ Wf[C * i:C * (i + 1)]).reshape(h, h, 16)
        r0, r1, a0, a1 = _upsample_weights(h, 224)
        rows = yi[r0] * a0[:, None, None] + yi[r1] * a1[:, None, None]
        yB = yB + rows[:, r0] * a0[None, :, None] + rows[:, r1] * a1[None, :, None]

    ybs = jax.ops.segment_sum(yB.reshape(-1, 16), idxs[0], num_segments=BLOCK_NUM)
    gf = _final_mm(adj, b0, ybs, cnt0, Wf[:C])

    finall = jnp.take(gf, idxs[0], axis=0).T.reshape(1, 16, 224, 224)
    sm = jax.nn.softmax(finall, axis=1)
    return finall, sm
